# baseline (speedup 1.0000x reference)
import jax
import jax.numpy as jnp
from jax import lax
from jax.experimental import pallas as pl
from jax.experimental.pallas import tpu as pltpu
from jax.experimental.pallas import tpu_sc as plsc

D = 1024
EPS = 1e-6
L = 16
NB = D // L
NW = 32
ROWS_PER_W = 512
C = 32
G = ROWS_PER_W // C
NBUF = 3
R = 8


def _lane_sum(x):
    lanes = lax.iota(jnp.int32, L)
    dn = lax.GatherDimensionNumbers(
        offset_dims=(), collapsed_slice_dims=(0,), start_index_map=(0,)
    )
    for sh in (8, 4, 2, 1):
        perm = lax.bitwise_xor(lanes, jnp.int32(sh))
        x = x + lax.gather(
            x,
            perm[:, None],
            dn,
            slice_sizes=(1,),
            mode=lax.GatherScatterMode.PROMISE_IN_BOUNDS,
        )
    return x


def _rsqrt_vec(x):
    i = lax.bitcast_convert_type(x, jnp.int32)
    i = jnp.int32(0x5F3759DF) - lax.shift_right_logical(i, 1)
    y = lax.bitcast_convert_type(i, jnp.float32)
    for _ in range(3):
        y = y * (1.5 - 0.5 * x * y * y)
    return y


def _body(table_h, idx_h, g_h, b_h, out_h, idx_v, rows_v, g_v, b_v, gsems, ssems):
    cid = lax.axis_index("c")
    sid = lax.axis_index("s")
    wid = sid * 2 + cid
    base = wid * ROWS_PER_W

    pltpu.sync_copy(idx_h.at[pl.ds(base, ROWS_PER_W)], idx_v)
    pltpu.sync_copy(g_h, g_v)
    pltpu.sync_copy(b_h, b_v)

    def gather_copy(g, b):
        row0 = pl.multiple_of(g * C, C)
        return pltpu.make_async_copy(
            table_h.at[idx_v.at[pl.ds(row0, C)]], rows_v.at[b], gsems.at[b]
        )

    def store_copy(g, b):
        row0 = pl.multiple_of(base + g * C, C)
        return pltpu.make_async_copy(
            rows_v.at[b], out_h.at[pl.ds(row0, C)], ssems.at[b]
        )

    def compute(b):
        buf = rows_v.at[b]

        def group_fn(rr, carry):
            r0 = rr * R
            acc = [jnp.zeros((L,), jnp.float32) for _ in range(R)]
            accsq = [jnp.zeros((L,), jnp.float32) for _ in range(R)]
            for j in range(NB):
                for q in range(R):
                    v = buf[r0 + q, pl.ds(j * L, L)]
                    acc[q] = acc[q] + v
                    accsq[q] = accsq[q] + v * v
            means = []
            rstds = []
            for q in range(R):
                mean_vec = _lane_sum(acc[q]) * (1.0 / D)
                var_vec = _lane_sum(accsq[q]) * (1.0 / D) - mean_vec * mean_vec
                means.append(mean_vec)
                rstds.append(_rsqrt_vec(var_vec + EPS))
            accy = [jnp.zeros((L,), jnp.float32) for _ in range(R)]
            for j in range(NB):
                gv = g_v[pl.ds(j * L, L)]
                bv = b_v[pl.ds(j * L, L)]
                for q in range(R):
                    v = buf[r0 + q, pl.ds(j * L, L)]
                    accy[q] = accy[q] + ((v - means[q]) * rstds[q] * gv + bv)
            for q in range(R):
                buf[r0 + q, pl.ds(0, L)] = accy[q]
            return carry

        lax.fori_loop(0, C // R, group_fn, 0)

    gather_copy(0, 0).start()

    def round_fn(t, carry):
        for b in range(NBUF):
            g = t * NBUF + b

            @pl.when(g < G)
            def _():
                b_next = (b + 1) % NBUF

                @pl.when(g >= 2)
                def _():
                    store_copy(g - 2, b_next).wait()

                @pl.when(g + 1 < G)
                def _():
                    gather_copy(g + 1, b_next).start()

                gather_copy(g, b).wait()
                compute(b)
                store_copy(g, b).start()

        return carry

    nrounds = (G + NBUF - 1) // NBUF
    lax.fori_loop(0, nrounds, round_fn, 0)

    store_copy(G - 2, (G - 2) % NBUF).wait()
    store_copy(G - 1, (G - 1) % NBUF).wait()


@jax.jit
def _emb_ln(table, idx, gamma, beta):
    mesh = plsc.VectorSubcoreMesh(core_axis_name="c", subcore_axis_name="s")
    return pl.kernel(
        _body,
        out_type=jax.ShapeDtypeStruct((idx.shape[0], D), jnp.float32),
        mesh=mesh,
        scratch_types=[
            pltpu.VMEM((ROWS_PER_W,), jnp.int32),
            pltpu.VMEM((NBUF, C, D), jnp.float32),
            pltpu.VMEM((D,), jnp.float32),
            pltpu.VMEM((D,), jnp.float32),
            pltpu.SemaphoreType.DMA((NBUF,)),
            pltpu.SemaphoreType.DMA((NBUF,)),
        ],
    )(table, idx, gamma, beta)


def kernel(src, table, gamma, beta):
    idx = src.reshape(-1).astype(jnp.int32)
    out = _emb_ln(table, idx, gamma, beta)
    return out.reshape(src.shape + (D,))


# speedup vs baseline: 1.2991x; 1.2991x over previous
import jax
import jax.numpy as jnp
from jax import lax
from jax.experimental import pallas as pl
from jax.experimental.pallas import tpu as pltpu
from jax.experimental.pallas import tpu_sc as plsc

D = 1024
EPS = 1e-6
L = 16
NB = D // L
NW = 32
ROWS_PER_W = 512
C = 32
G = ROWS_PER_W // C
NBUF = 3
R = 8


def _lane_sum(x):
    lanes = lax.iota(jnp.int32, L)
    dn = lax.GatherDimensionNumbers(
        offset_dims=(), collapsed_slice_dims=(0,), start_index_map=(0,)
    )
    for sh in (8, 4, 2, 1):
        perm = lax.bitwise_xor(lanes, jnp.int32(sh))
        x = x + lax.gather(
            x,
            perm[:, None],
            dn,
            slice_sizes=(1,),
            mode=lax.GatherScatterMode.PROMISE_IN_BOUNDS,
        )
    return x


def _rsqrt_vec(x):
    i = lax.bitcast_convert_type(x, jnp.int32)
    i = jnp.int32(0x5F3759DF) - lax.shift_right_logical(i, 1)
    y = lax.bitcast_convert_type(i, jnp.float32)
    for _ in range(3):
        y = y * (1.5 - 0.5 * x * y * y)
    return y


def _body(table_h, idx_h, g_h, b_h, out_h, idx_v, rows_v, g_v, b_v, gsems, ssems):
    cid = lax.axis_index("c")
    sid = lax.axis_index("s")
    wid = sid * 2 + cid
    base = wid * ROWS_PER_W

    pltpu.sync_copy(idx_h.at[pl.ds(base, ROWS_PER_W)], idx_v)
    pltpu.sync_copy(g_h, g_v)
    pltpu.sync_copy(b_h, b_v)

    def gather_copy(g, b):
        row0 = pl.multiple_of(g * C, C)
        return pltpu.make_async_copy(
            table_h.at[idx_v.at[pl.ds(row0, C)]], rows_v.at[b], gsems.at[b]
        )

    def store_copy(g, b):
        row0 = pl.multiple_of(base + g * C, C)
        return pltpu.make_async_copy(
            rows_v.at[b], out_h.at[pl.ds(row0, C)], ssems.at[b]
        )

    def compute(b):
        buf = rows_v.at[b]

        def group_fn(rr, carry):
            r0 = rr * R
            acc = [jnp.zeros((L,), jnp.float32) for _ in range(R)]
            accsq = [jnp.zeros((L,), jnp.float32) for _ in range(R)]
            for j in range(NB):
                for q in range(R):
                    v = buf[r0 + q, pl.ds(j * L, L)]
                    acc[q] = acc[q] + v
                    accsq[q] = accsq[q] + v * v
            means = []
            rstds = []
            for q in range(R):
                mean_vec = _lane_sum(acc[q]) * (1.0 / D)
                var_vec = _lane_sum(accsq[q]) * (1.0 / D) - mean_vec * mean_vec
                means.append(mean_vec)
                rstds.append(_rsqrt_vec(var_vec + EPS))
            JB = 4
            RT = 4
            for rt in range(R // RT):
                for jt in range(NB // JB):
                    gvs = [g_v[pl.ds((jt * JB + k) * L, L)] for k in range(JB)]
                    bvs = [b_v[pl.ds((jt * JB + k) * L, L)] for k in range(JB)]
                    vs = []
                    for q in range(RT):
                        row = rt * RT + q
                        vs.append([
                            buf[r0 + row, pl.ds((jt * JB + k) * L, L)]
                            for k in range(JB)
                        ])
                    for q in range(RT):
                        row = rt * RT + q
                        for k in range(JB):
                            y = (vs[q][k] - means[row]) * rstds[row]
                            buf[r0 + row, pl.ds((jt * JB + k) * L, L)] = (
                                y * gvs[k] + bvs[k]
                            )
            return carry

        lax.fori_loop(0, C // R, group_fn, 0)

    gather_copy(0, 0).start()

    def round_fn(t, carry):
        for b in range(NBUF):
            g = t * NBUF + b

            @pl.when(g < G)
            def _():
                b_next = (b + 1) % NBUF

                @pl.when(g >= 2)
                def _():
                    store_copy(g - 2, b_next).wait()

                @pl.when(g + 1 < G)
                def _():
                    gather_copy(g + 1, b_next).start()

                gather_copy(g, b).wait()
                compute(b)
                store_copy(g, b).start()

        return carry

    nrounds = (G + NBUF - 1) // NBUF
    lax.fori_loop(0, nrounds, round_fn, 0)

    store_copy(G - 2, (G - 2) % NBUF).wait()
    store_copy(G - 1, (G - 1) % NBUF).wait()


@jax.jit
def _emb_ln(table, idx, gamma, beta):
    mesh = plsc.VectorSubcoreMesh(core_axis_name="c", subcore_axis_name="s")
    return pl.kernel(
        _body,
        out_type=jax.ShapeDtypeStruct((idx.shape[0], D), jnp.float32),
        mesh=mesh,
        scratch_types=[
            pltpu.VMEM((ROWS_PER_W,), jnp.int32),
            pltpu.VMEM((NBUF, C, D), jnp.float32),
            pltpu.VMEM((D,), jnp.float32),
            pltpu.VMEM((D,), jnp.float32),
            pltpu.SemaphoreType.DMA((NBUF,)),
            pltpu.SemaphoreType.DMA((NBUF,)),
        ],
    )(table, idx, gamma, beta)


def kernel(src, table, gamma, beta):
    idx = src.reshape(-1).astype(jnp.int32)
    out = _emb_ln(table, idx, gamma, beta)
    return out.reshape(src.shape + (D,))


# speedup vs baseline: 1.9828x; 1.5263x over previous
import jax
import jax.numpy as jnp
from jax import lax
from jax.experimental import pallas as pl
from jax.experimental.pallas import tpu as pltpu
from jax.experimental.pallas import tpu_sc as plsc

D = 1024
EPS = 1e-6
NW = 32
NTOK = 16384
K = 4
CH = NTOK // K
ROWS_PER_W = CH // NW
C = 32
G = ROWS_PER_W // C
NBUF = 3
BR = 256


def _gather_body(table_h, idx_h, out_h, idx_v, rows_v, gsems, ssems):
    cid = lax.axis_index("c")
    sid = lax.axis_index("s")
    wid = sid * 2 + cid
    base = wid * ROWS_PER_W

    pltpu.sync_copy(idx_h.at[pl.ds(base, ROWS_PER_W)], idx_v)

    def gather_copy(g, b):
        return pltpu.make_async_copy(
            table_h.at[idx_v.at[pl.ds(g * C, C)]], rows_v.at[b], gsems.at[b]
        )

    def store_copy(g, b):
        return pltpu.make_async_copy(
            rows_v.at[b], out_h.at[pl.ds(base + g * C, C)], ssems.at[b]
        )

    gather_copy(0, 0).start()
    for g in range(G):
        b = g % NBUF
        if g >= 2:
            store_copy(g - 2, (g - 2) % NBUF).wait()
        if g + 1 < G:
            gather_copy(g + 1, (g + 1) % NBUF).start()
        gather_copy(g, b).wait()
        store_copy(g, b).start()
    for g in range(max(G - 2, 0), G):
        store_copy(g, g % NBUF).wait()


def _sc_gather(table, idx_chunk):
    mesh = plsc.VectorSubcoreMesh(core_axis_name="c", subcore_axis_name="s")
    return pl.kernel(
        _gather_body,
        out_type=jax.ShapeDtypeStruct((CH, D), jnp.float32),
        mesh=mesh,
        scratch_types=[
            pltpu.VMEM((ROWS_PER_W,), jnp.int32),
            pltpu.VMEM((NBUF, C, D), jnp.float32),
            pltpu.SemaphoreType.DMA((NBUF,)),
            pltpu.SemaphoreType.DMA((NBUF,)),
        ],
    )(table, idx_chunk)


def _ln_body(x_ref, g_ref, b_ref, o_ref):
    x = x_ref[...]
    m = jnp.mean(x, axis=-1, keepdims=True)
    xc = x - m
    var = jnp.mean(xc * xc, axis=-1, keepdims=True)
    o_ref[...] = xc * lax.rsqrt(var + EPS) * g_ref[...] + b_ref[...]


def _tc_ln(x, gamma, beta):
    return pl.pallas_call(
        _ln_body,
        grid=(CH // BR,),
        in_specs=[
            pl.BlockSpec((BR, D), lambda i: (i, 0)),
            pl.BlockSpec((D,), lambda i: (0,)),
            pl.BlockSpec((D,), lambda i: (0,)),
        ],
        out_specs=pl.BlockSpec((BR, D), lambda i: (i, 0)),
        out_shape=jax.ShapeDtypeStruct((CH, D), jnp.float32),
    )(x, gamma, beta)


@jax.jit
def _emb_ln(table, idx, gamma, beta):
    idx_chunks = idx.reshape(K, CH)
    outs = []
    for k in range(K):
        gathered = _sc_gather(table, idx_chunks[k])
        outs.append(_tc_ln(gathered, gamma, beta))
    return jnp.concatenate(outs, axis=0)


def kernel(src, table, gamma, beta):
    idx = src.reshape(-1).astype(jnp.int32)
    out = _emb_ln(table, idx, gamma, beta)
    return out.reshape(src.shape + (D,))


# speedup vs baseline: 1.9875x; 1.0023x over previous
import jax
import jax.numpy as jnp
from jax import lax
from jax.experimental import pallas as pl
from jax.experimental.pallas import tpu as pltpu
from jax.experimental.pallas import tpu_sc as plsc

D = 1024
EPS = 1e-6
NW = 32
NTOK = 16384
K = 4
CH = NTOK // K
ROWS_PER_W = CH // NW
C = 32
G = ROWS_PER_W // C
NBUF = 3
BR = 256


def _gather_body(table_h, idx_h, out_h, idx_v, rows_v, gsems, ssems):
    cid = lax.axis_index("c")
    sid = lax.axis_index("s")
    wid = sid * 2 + cid
    base = wid * ROWS_PER_W

    pltpu.sync_copy(idx_h.at[pl.ds(base, ROWS_PER_W)], idx_v)

    def gather_copy(g, b):
        return pltpu.make_async_copy(
            table_h.at[idx_v.at[pl.ds(g * C, C)]], rows_v.at[b], gsems.at[b]
        )

    def store_copy(g, b):
        return pltpu.make_async_copy(
            rows_v.at[b], out_h.at[pl.ds(base + g * C, C)], ssems.at[b]
        )

    gather_copy(0, 0).start()
    for g in range(G):
        b = g % NBUF
        if g >= 2:
            store_copy(g - 2, (g - 2) % NBUF).wait()
        if g + 1 < G:
            gather_copy(g + 1, (g + 1) % NBUF).start()
        gather_copy(g, b).wait()
        store_copy(g, b).start()
    for g in range(max(G - 2, 0), G):
        store_copy(g, g % NBUF).wait()


def _sc_gather(table, idx_chunk):
    mesh = plsc.VectorSubcoreMesh(core_axis_name="c", subcore_axis_name="s")
    return pl.kernel(
        _gather_body,
        out_type=jax.ShapeDtypeStruct((CH, D), jnp.float32),
        mesh=mesh,
        scratch_types=[
            pltpu.VMEM((ROWS_PER_W,), jnp.int32),
            pltpu.VMEM((NBUF, C, D), jnp.float32),
            pltpu.SemaphoreType.DMA((NBUF,)),
            pltpu.SemaphoreType.DMA((NBUF,)),
        ],
    )(table, idx_chunk)


def _ln_body(x_ref, g_ref, b_ref, o_ref):
    x = x_ref[...]
    m = jnp.mean(x, axis=-1, keepdims=True)
    xc = x - m
    var = jnp.mean(xc * xc, axis=-1, keepdims=True)
    o_ref[...] = xc * lax.rsqrt(var + EPS) * g_ref[...] + b_ref[...]


def _tc_ln(x, gamma, beta):
    return pl.pallas_call(
        _ln_body,
        grid=(CH // BR,),
        in_specs=[
            pl.BlockSpec((BR, D), lambda i: (i, 0)),
            pl.BlockSpec((D,), lambda i: (0,)),
            pl.BlockSpec((D,), lambda i: (0,)),
        ],
        out_specs=pl.BlockSpec((BR, D), lambda i: (i, 0)),
        out_shape=jax.ShapeDtypeStruct((CH, D), jnp.float32),
    )(x, gamma, beta)


@jax.jit
def _emb_ln(table, idx, gamma, beta):
    idx_chunks = idx.reshape(K, CH)
    gathered = [_sc_gather(table, idx_chunks[k]) for k in range(K)]
    outs = [_tc_ln(g, gamma, beta) for g in gathered]
    return jnp.concatenate(outs, axis=0)


def kernel(src, table, gamma, beta):
    idx = src.reshape(-1).astype(jnp.int32)
    out = _emb_ln(table, idx, gamma, beta)
    return out.reshape(src.shape + (D,))


# speedup vs baseline: 2.5049x; 1.2603x over previous
import jax
import jax.numpy as jnp
from jax import lax
from jax.experimental import pallas as pl
from jax.experimental.pallas import tpu as pltpu
from jax.experimental.pallas import tpu_sc as plsc

D = 1024
EPS = 1e-6
NW = 32
NTOK = 16384
K = 1
CH = NTOK // K
ROWS_PER_W = CH // NW
C = 32
G = ROWS_PER_W // C
NBUF = 3
BR = 256


def _gather_body(table_h, idx_h, out_h, idx_v, rows_v, gsems, ssems):
    cid = lax.axis_index("c")
    sid = lax.axis_index("s")
    wid = sid * 2 + cid
    base = wid * ROWS_PER_W

    pltpu.sync_copy(idx_h.at[pl.ds(base, ROWS_PER_W)], idx_v)

    def gather_copy(g, b):
        return pltpu.make_async_copy(
            table_h.at[idx_v.at[pl.ds(g * C, C)]], rows_v.at[b], gsems.at[b]
        )

    def store_copy(g, b):
        return pltpu.make_async_copy(
            rows_v.at[b], out_h.at[pl.ds(base + g * C, C)], ssems.at[b]
        )

    gather_copy(0, 0).start()
    for g in range(G):
        b = g % NBUF
        if g >= 2:
            store_copy(g - 2, (g - 2) % NBUF).wait()
        if g + 1 < G:
            gather_copy(g + 1, (g + 1) % NBUF).start()
        gather_copy(g, b).wait()
        store_copy(g, b).start()
    for g in range(max(G - 2, 0), G):
        store_copy(g, g % NBUF).wait()


def _sc_gather(table, idx_chunk):
    mesh = plsc.VectorSubcoreMesh(core_axis_name="c", subcore_axis_name="s")
    return pl.kernel(
        _gather_body,
        out_type=jax.ShapeDtypeStruct((CH, D), jnp.float32),
        mesh=mesh,
        scratch_types=[
            pltpu.VMEM((ROWS_PER_W,), jnp.int32),
            pltpu.VMEM((NBUF, C, D), jnp.float32),
            pltpu.SemaphoreType.DMA((NBUF,)),
            pltpu.SemaphoreType.DMA((NBUF,)),
        ],
    )(table, idx_chunk)


def _ln_body(x_ref, g_ref, b_ref, o_ref):
    x = x_ref[...]
    m = jnp.mean(x, axis=-1, keepdims=True)
    xc = x - m
    var = jnp.mean(xc * xc, axis=-1, keepdims=True)
    o_ref[...] = xc * lax.rsqrt(var + EPS) * g_ref[...] + b_ref[...]


def _tc_ln(x, gamma, beta):
    return pl.pallas_call(
        _ln_body,
        grid=(CH // BR,),
        in_specs=[
            pl.BlockSpec((BR, D), lambda i: (i, 0)),
            pl.BlockSpec((D,), lambda i: (0,)),
            pl.BlockSpec((D,), lambda i: (0,)),
        ],
        out_specs=pl.BlockSpec((BR, D), lambda i: (i, 0)),
        out_shape=jax.ShapeDtypeStruct((CH, D), jnp.float32),
    )(x, gamma, beta)


@jax.jit
def _emb_ln(table, idx, gamma, beta):
    idx_chunks = idx.reshape(K, CH)
    gathered = [_sc_gather(table, idx_chunks[k]) for k in range(K)]
    outs = [_tc_ln(g, gamma, beta) for g in gathered]
    return jnp.concatenate(outs, axis=0)


def kernel(src, table, gamma, beta):
    idx = src.reshape(-1).astype(jnp.int32)
    out = _emb_ln(table, idx, gamma, beta)
    return out.reshape(src.shape + (D,))


# speedup vs baseline: 2.9481x; 1.1769x over previous
import jax
import jax.numpy as jnp
from jax import lax
from jax.experimental import pallas as pl
from jax.experimental.pallas import tpu as pltpu
from jax.experimental.pallas import tpu_sc as plsc

D = 1024
EPS = 1e-6
NW = 32
NTOK = 16384
K = 1
CH = NTOK // K
ROWS_PER_W = CH // NW
C = 32
G = ROWS_PER_W // C
NBUF = 3
BR = 512


def _gather_body(table_h, idx_h, out_h, idx_v, rows_v, gsems, ssems):
    cid = lax.axis_index("c")
    sid = lax.axis_index("s")
    wid = sid * 2 + cid
    base = wid * ROWS_PER_W

    pltpu.sync_copy(idx_h.at[pl.ds(base, ROWS_PER_W)], idx_v)

    def gather_copy(g, b):
        return pltpu.make_async_copy(
            table_h.at[idx_v.at[pl.ds(g * C, C)]], rows_v.at[b], gsems.at[b]
        )

    def store_copy(g, b):
        return pltpu.make_async_copy(
            rows_v.at[b], out_h.at[pl.ds(base + g * C, C)], ssems.at[b]
        )

    gather_copy(0, 0).start()
    for g in range(G):
        b = g % NBUF
        if g >= 2:
            store_copy(g - 2, (g - 2) % NBUF).wait()
        if g + 1 < G:
            gather_copy(g + 1, (g + 1) % NBUF).start()
        gather_copy(g, b).wait()
        store_copy(g, b).start()
    for g in range(max(G - 2, 0), G):
        store_copy(g, g % NBUF).wait()


def _sc_gather(table, idx_chunk):
    mesh = plsc.VectorSubcoreMesh(core_axis_name="c", subcore_axis_name="s")
    return pl.kernel(
        _gather_body,
        out_type=jax.ShapeDtypeStruct((CH, D), jnp.float32),
        mesh=mesh,
        scratch_types=[
            pltpu.VMEM((ROWS_PER_W,), jnp.int32),
            pltpu.VMEM((NBUF, C, D), jnp.float32),
            pltpu.SemaphoreType.DMA((NBUF,)),
            pltpu.SemaphoreType.DMA((NBUF,)),
        ],
    )(table, idx_chunk)


def _ln_body(x_ref, g_ref, b_ref, o_ref):
    x = x_ref[...]
    m = jnp.mean(x, axis=-1, keepdims=True)
    xc = x - m
    var = jnp.mean(xc * xc, axis=-1, keepdims=True)
    o_ref[...] = xc * lax.rsqrt(var + EPS) * g_ref[...] + b_ref[...]


def _tc_ln(x, gamma, beta):
    return pl.pallas_call(
        _ln_body,
        grid=(CH // BR,),
        in_specs=[
            pl.BlockSpec((BR, D), lambda i: (i, 0)),
            pl.BlockSpec((D,), lambda i: (0,)),
            pl.BlockSpec((D,), lambda i: (0,)),
        ],
        out_specs=pl.BlockSpec((BR, D), lambda i: (i, 0)),
        out_shape=jax.ShapeDtypeStruct((CH, D), jnp.float32),
    )(x, gamma, beta)


@jax.jit
def _emb_ln(table, idx, gamma, beta):
    idx_chunks = idx.reshape(K, CH)
    gathered = [_sc_gather(table, idx_chunks[k]) for k in range(K)]
    outs = [_tc_ln(g, gamma, beta) for g in gathered]
    return jnp.concatenate(outs, axis=0)


def kernel(src, table, gamma, beta):
    idx = src.reshape(-1).astype(jnp.int32)
    out = _emb_ln(table, idx, gamma, beta)
    return out.reshape(src.shape + (D,))


# speedup vs baseline: 3.1212x; 1.0587x over previous
import jax
import jax.numpy as jnp
from jax import lax
from jax.experimental import pallas as pl
from jax.experimental.pallas import tpu as pltpu
from jax.experimental.pallas import tpu_sc as plsc

D = 1024
EPS = 1e-6
NW = 32
NTOK = 16384
K = 1
CH = NTOK // K
ROWS_PER_W = CH // NW
C = 32
G = ROWS_PER_W // C
NBUF = 3
BR = 1024


def _gather_body(table_h, idx_h, out_h, idx_v, rows_v, gsems, ssems):
    cid = lax.axis_index("c")
    sid = lax.axis_index("s")
    wid = sid * 2 + cid
    base = wid * ROWS_PER_W

    pltpu.sync_copy(idx_h.at[pl.ds(base, ROWS_PER_W)], idx_v)

    def gather_copy(g, b):
        return pltpu.make_async_copy(
            table_h.at[idx_v.at[pl.ds(g * C, C)]], rows_v.at[b], gsems.at[b]
        )

    def store_copy(g, b):
        return pltpu.make_async_copy(
            rows_v.at[b], out_h.at[pl.ds(base + g * C, C)], ssems.at[b]
        )

    gather_copy(0, 0).start()
    for g in range(G):
        b = g % NBUF
        if g >= 2:
            store_copy(g - 2, (g - 2) % NBUF).wait()
        if g + 1 < G:
            gather_copy(g + 1, (g + 1) % NBUF).start()
        gather_copy(g, b).wait()
        store_copy(g, b).start()
    for g in range(max(G - 2, 0), G):
        store_copy(g, g % NBUF).wait()


def _sc_gather(table, idx_chunk):
    mesh = plsc.VectorSubcoreMesh(core_axis_name="c", subcore_axis_name="s")
    return pl.kernel(
        _gather_body,
        out_type=jax.ShapeDtypeStruct((CH, D), jnp.float32),
        mesh=mesh,
        scratch_types=[
            pltpu.VMEM((ROWS_PER_W,), jnp.int32),
            pltpu.VMEM((NBUF, C, D), jnp.float32),
            pltpu.SemaphoreType.DMA((NBUF,)),
            pltpu.SemaphoreType.DMA((NBUF,)),
        ],
    )(table, idx_chunk)


def _ln_body(x_ref, g_ref, b_ref, o_ref):
    x = x_ref[...]
    m = jnp.mean(x, axis=-1, keepdims=True)
    xc = x - m
    var = jnp.mean(xc * xc, axis=-1, keepdims=True)
    o_ref[...] = xc * lax.rsqrt(var + EPS) * g_ref[...] + b_ref[...]


def _tc_ln(x, gamma, beta):
    return pl.pallas_call(
        _ln_body,
        grid=(CH // BR,),
        in_specs=[
            pl.BlockSpec((BR, D), lambda i: (i, 0)),
            pl.BlockSpec((D,), lambda i: (0,)),
            pl.BlockSpec((D,), lambda i: (0,)),
        ],
        out_specs=pl.BlockSpec((BR, D), lambda i: (i, 0)),
        out_shape=jax.ShapeDtypeStruct((CH, D), jnp.float32),
    )(x, gamma, beta)


@jax.jit
def _emb_ln(table, idx, gamma, beta):
    idx_chunks = idx.reshape(K, CH)
    gathered = [_sc_gather(table, idx_chunks[k]) for k in range(K)]
    outs = [_tc_ln(g, gamma, beta) for g in gathered]
    return jnp.concatenate(outs, axis=0)


def kernel(src, table, gamma, beta):
    idx = src.reshape(-1).astype(jnp.int32)
    out = _emb_ln(table, idx, gamma, beta)
    return out.reshape(src.shape + (D,))


# speedup vs baseline: 3.1524x; 1.0100x over previous
import jax
import jax.numpy as jnp
from jax import lax
from jax.experimental import pallas as pl
from jax.experimental.pallas import tpu as pltpu
from jax.experimental.pallas import tpu_sc as plsc

D = 1024
EPS = 1e-6
NW = 32
NTOK = 16384
K = 1
CH = NTOK // K
ROWS_PER_W = CH // NW
C = 32
G = ROWS_PER_W // C
NBUF = 3
BR = 2048


def _gather_body(table_h, idx_h, out_h, idx_v, rows_v, gsems, ssems):
    cid = lax.axis_index("c")
    sid = lax.axis_index("s")
    wid = sid * 2 + cid
    base = wid * ROWS_PER_W

    pltpu.sync_copy(idx_h.at[pl.ds(base, ROWS_PER_W)], idx_v)

    def gather_copy(g, b):
        return pltpu.make_async_copy(
            table_h.at[idx_v.at[pl.ds(g * C, C)]], rows_v.at[b], gsems.at[b]
        )

    def store_copy(g, b):
        return pltpu.make_async_copy(
            rows_v.at[b], out_h.at[pl.ds(base + g * C, C)], ssems.at[b]
        )

    gather_copy(0, 0).start()
    for g in range(G):
        b = g % NBUF
        if g >= 2:
            store_copy(g - 2, (g - 2) % NBUF).wait()
        if g + 1 < G:
            gather_copy(g + 1, (g + 1) % NBUF).start()
        gather_copy(g, b).wait()
        store_copy(g, b).start()
    for g in range(max(G - 2, 0), G):
        store_copy(g, g % NBUF).wait()


def _sc_gather(table, idx_chunk):
    mesh = plsc.VectorSubcoreMesh(core_axis_name="c", subcore_axis_name="s")
    return pl.kernel(
        _gather_body,
        out_type=jax.ShapeDtypeStruct((CH, D), jnp.float32),
        mesh=mesh,
        scratch_types=[
            pltpu.VMEM((ROWS_PER_W,), jnp.int32),
            pltpu.VMEM((NBUF, C, D), jnp.float32),
            pltpu.SemaphoreType.DMA((NBUF,)),
            pltpu.SemaphoreType.DMA((NBUF,)),
        ],
    )(table, idx_chunk)


def _ln_body(x_ref, g_ref, b_ref, o_ref):
    x = x_ref[...]
    m = jnp.mean(x, axis=-1, keepdims=True)
    xc = x - m
    var = jnp.mean(xc * xc, axis=-1, keepdims=True)
    o_ref[...] = xc * lax.rsqrt(var + EPS) * g_ref[...] + b_ref[...]


def _tc_ln(x, gamma, beta):
    return pl.pallas_call(
        _ln_body,
        grid=(CH // BR,),
        in_specs=[
            pl.BlockSpec((BR, D), lambda i: (i, 0)),
            pl.BlockSpec((D,), lambda i: (0,)),
            pl.BlockSpec((D,), lambda i: (0,)),
        ],
        out_specs=pl.BlockSpec((BR, D), lambda i: (i, 0)),
        out_shape=jax.ShapeDtypeStruct((CH, D), jnp.float32),
    )(x, gamma, beta)


@jax.jit
def _emb_ln(table, idx, gamma, beta):
    idx_chunks = idx.reshape(K, CH)
    gathered = [_sc_gather(table, idx_chunks[k]) for k in range(K)]
    outs = [_tc_ln(g, gamma, beta) for g in gathered]
    return jnp.concatenate(outs, axis=0)


def kernel(src, table, gamma, beta):
    idx = src.reshape(-1).astype(jnp.int32)
    out = _emb_ln(table, idx, gamma, beta)
    return out.reshape(src.shape + (D,))


# speedup vs baseline: 3.1809x; 1.0091x over previous
import jax
import jax.numpy as jnp
from jax import lax
from jax.experimental import pallas as pl
from jax.experimental.pallas import tpu as pltpu
from jax.experimental.pallas import tpu_sc as plsc

D = 1024
EPS = 1e-6
NW = 32
NTOK = 16384
K = 1
CH = NTOK // K
ROWS_PER_W = CH // NW
C = 16
G = ROWS_PER_W // C
NBUF = 4
BR = 2048


def _gather_body(table_h, idx_h, out_h, idx_v, rows_v, gsems, ssems):
    cid = lax.axis_index("c")
    sid = lax.axis_index("s")
    wid = sid * 2 + cid
    base = wid * ROWS_PER_W

    pltpu.sync_copy(idx_h.at[pl.ds(base, ROWS_PER_W)], idx_v)

    def gather_copy(g, b):
        return pltpu.make_async_copy(
            table_h.at[idx_v.at[pl.ds(g * C, C)]], rows_v.at[b], gsems.at[b]
        )

    def store_copy(g, b):
        return pltpu.make_async_copy(
            rows_v.at[b], out_h.at[pl.ds(base + g * C, C)], ssems.at[b]
        )

    gather_copy(0, 0).start()
    gather_copy(1, 1).start()
    for g in range(G):
        b = g % NBUF
        if g >= 2:
            store_copy(g - 2, (g - 2) % NBUF).wait()
        if g + 2 < G:
            gather_copy(g + 2, (g + 2) % NBUF).start()
        gather_copy(g, b).wait()
        store_copy(g, b).start()
    for g in range(max(G - 2, 0), G):
        store_copy(g, g % NBUF).wait()


def _sc_gather(table, idx_chunk):
    mesh = plsc.VectorSubcoreMesh(core_axis_name="c", subcore_axis_name="s")
    return pl.kernel(
        _gather_body,
        out_type=jax.ShapeDtypeStruct((CH, D), jnp.float32),
        mesh=mesh,
        scratch_types=[
            pltpu.VMEM((ROWS_PER_W,), jnp.int32),
            pltpu.VMEM((NBUF, C, D), jnp.float32),
            pltpu.SemaphoreType.DMA((NBUF,)),
            pltpu.SemaphoreType.DMA((NBUF,)),
        ],
    )(table, idx_chunk)


def _ln_body(x_ref, g_ref, b_ref, o_ref):
    x = x_ref[...]
    m = jnp.mean(x, axis=-1, keepdims=True)
    xc = x - m
    var = jnp.mean(xc * xc, axis=-1, keepdims=True)
    o_ref[...] = xc * lax.rsqrt(var + EPS) * g_ref[...] + b_ref[...]


def _tc_ln(x, gamma, beta):
    return pl.pallas_call(
        _ln_body,
        grid=(CH // BR,),
        in_specs=[
            pl.BlockSpec((BR, D), lambda i: (i, 0)),
            pl.BlockSpec((D,), lambda i: (0,)),
            pl.BlockSpec((D,), lambda i: (0,)),
        ],
        out_specs=pl.BlockSpec((BR, D), lambda i: (i, 0)),
        out_shape=jax.ShapeDtypeStruct((CH, D), jnp.float32),
    )(x, gamma, beta)


@jax.jit
def _emb_ln(table, idx, gamma, beta):
    idx_chunks = idx.reshape(K, CH)
    gathered = [_sc_gather(table, idx_chunks[k]) for k in range(K)]
    outs = [_tc_ln(g, gamma, beta) for g in gathered]
    return jnp.concatenate(outs, axis=0)


def kernel(src, table, gamma, beta):
    idx = src.reshape(-1).astype(jnp.int32)
    out = _emb_ln(table, idx, gamma, beta)
    return out.reshape(src.shape + (D,))
